# baseline (speedup 1.0000x reference)
import jax
import jax.numpy as jnp
from jax import lax
from jax.experimental import pallas as pl
from jax.experimental.pallas import tpu as pltpu
from jax.experimental.pallas import tpu_sc as plsc

N_NODES_ = 10000
CH_ = 128
R_ = 10240
ROWS_PER_TILE_ = R_ // 16
E_ = 128
ADJ0_, ADJ1_, NB_ADJ_ = 48, 112, 8
FEAT0_, FEAT1_, NB_FEAT_ = 15, 35, 5
TOTCH_ADJ_ = 16 * (ADJ0_ + ADJ1_)
TOTCH_FEAT_ = 16 * (FEAT0_ + FEAT1_)
PADJ_ = TOTCH_ADJ_ * E_
PFEAT_ = TOTCH_FEAT_ * E_

_GDN = lax.GatherDimensionNumbers(
    offset_dims=(), collapsed_slice_dims=(0,), start_index_map=(0,))


def _lane_bcast(vv, zidx, i):
    return lax.gather(vv, zidx + i, _GDN, slice_sizes=(1,),
                      mode=lax.GatherScatterMode.PROMISE_IN_BOUNDS)


def _make_scatter_body(nch0, nch1, nb):
    def body(dst_h, src_h, val_h, table_h, part_h,
             acc, dstb, srcb, valb, rows0, rows1, sem0, sem1):
        rows_bufs = (rows0, rows1)
        sems = (sem0, sem1)
        c = lax.axis_index("c")
        s = lax.axis_index("s")
        cR = c * R_
        slab0 = s * ROWS_PER_TILE_
        zidx = jnp.reshape(lax.iota(jnp.int32, 16) * 0, (16, 1))
        zv = jnp.zeros((16,), jnp.float32)
        nbat = jnp.where(c == 0, nch0 // nb, nch1 // nb)
        batch0 = jnp.where(c == 0, 0, 16 * (nch0 // nb)) + s * nbat

        def _zb(i, carry):
            for j in range(8):
                rows0[i, pl.ds(j * 16, 16)] = zv
            return carry

        with jax.named_scope("zero_acc"):
            lax.fori_loop(0, E_, _zb, 0)
            for b in range(ROWS_PER_TILE_ // E_):
                pltpu.sync_copy(rows0, acc.at[pl.ds(slab0 + b * E_, E_)])
            plsc.subcore_barrier()

        def _batch(bi, carry):
            bidx = batch0 + bi
            e0 = bidx * (nb * E_)
            pltpu.sync_copy(dst_h.at[bidx], dstb)
            pltpu.sync_copy(src_h.at[pl.ds(e0, nb * E_)], srcb)
            pltpu.sync_copy(val_h.at[pl.ds(e0, nb * E_)], valb)
            h = pltpu.async_copy(table_h.at[srcb.at[pl.ds(0, E_)]],
                                 rows_bufs[0], sems[0])
            for kk in range(nb):
                rows = rows_bufs[kk % 2]
                if kk + 1 < nb:
                    idx_view = srcb.at[pl.ds((kk + 1) * E_, E_)]
                    h_next = pltpu.async_copy(table_h.at[idx_view],
                                              rows_bufs[(kk + 1) % 2],
                                              sems[(kk + 1) % 2])
                h.wait()
                if kk + 1 < nb:
                    h = h_next

                def _scale(g, c2, _kk=kk, _rows=rows):
                    vv = valb[pl.ds(_kk * E_ + g * 16, 16)]
                    for i in range(16):
                        bv = _lane_bcast(vv, zidx, i)
                        e = g * 16 + i
                        for j in range(8):
                            sl = pl.ds(j * 16, 16)
                            _rows[e, sl] = _rows[e, sl] * bv
                    return c2

                lax.fori_loop(0, E_ // 16, _scale, 0)
                pltpu.sync_copy(rows, acc.at[dstb.at[kk]], add=True)
            return carry

        with jax.named_scope("edge_stream"):
            lax.fori_loop(0, nbat, _batch, 0)
        with jax.named_scope("post_barrier"):
            plsc.subcore_barrier()

        with jax.named_scope("dump"):
            pltpu.sync_copy(acc.at[pl.ds(slab0, ROWS_PER_TILE_)],
                            part_h.at[pl.ds(cR + slab0, ROWS_PER_TILE_)])

    return body


def _make_scatter_call(nch0, nch1, nb):
    return pl.kernel(
        _make_scatter_body(nch0, nch1, nb),
        mesh=plsc.VectorSubcoreMesh(core_axis_name="c", subcore_axis_name="s"),
        out_type=jax.ShapeDtypeStruct((2 * R_, CH_), jnp.float32),
        scratch_types=[
            pltpu.VMEM_SHARED((R_, CH_), jnp.float32),
            pltpu.VMEM((nb, E_), jnp.int32),
            pltpu.VMEM((nb * E_,), jnp.int32),
            pltpu.VMEM((nb * E_,), jnp.float32),
            pltpu.VMEM((E_, CH_), jnp.float32),
            pltpu.VMEM((E_, CH_), jnp.float32),
            pltpu.SemaphoreType.DMA,
            pltpu.SemaphoreType.DMA,
        ],
    )


_scatter_feat = _make_scatter_call(FEAT0_, FEAT1_, NB_FEAT_)
_scatter_adj = _make_scatter_call(ADJ0_, ADJ1_, NB_ADJ_)

_BLK_ = 1280


def _combine_relu_body(p0, p1, b, o):
    o[...] = jnp.maximum(p0[...] + p1[...] + b[...], 0.0)


def _combine_body(p0, p1, o):
    o[...] = p0[...] + p1[...]


def _combine(part, bias=None):
    grid = (R_ // _BLK_,)
    spec0 = pl.BlockSpec((_BLK_, CH_), lambda i: (i, 0))
    spec1 = pl.BlockSpec((_BLK_, CH_), lambda i: (R_ // _BLK_ + i, 0))
    if bias is not None:
        return pl.pallas_call(
            _combine_relu_body,
            grid=grid,
            in_specs=[spec0, spec1, pl.BlockSpec((1, CH_), lambda i: (0, 0))],
            out_specs=pl.BlockSpec((_BLK_, CH_), lambda i: (i, 0)),
            out_shape=jax.ShapeDtypeStruct((R_, CH_), jnp.float32),
        )(part, part, bias)
    return pl.pallas_call(
        _combine_body,
        grid=grid,
        in_specs=[spec0, spec1],
        out_specs=pl.BlockSpec((_BLK_, CH_), lambda i: (i, 0)),
        out_shape=jax.ShapeDtypeStruct((R_, CH_), jnp.float32),
    )(part, part)


def _pad_to(x, n, fill=0):
    return jnp.concatenate([x, jnp.full((n - x.shape[0],), fill, x.dtype)])


@jax.jit
def kernel(adj_indices, adj_values, feat_row, feat_col, feat_values,
           weight, bias):
    adj_dst = _pad_to(adj_indices[0].astype(jnp.int32), PADJ_)
    adj_src = _pad_to(adj_indices[1].astype(jnp.int32), PADJ_)
    adj_val = _pad_to(adj_values.astype(jnp.float32), PADJ_)
    f_dst = _pad_to(feat_row.astype(jnp.int32), PFEAT_)
    f_src = _pad_to(feat_col.astype(jnp.int32), PFEAT_)
    f_val = _pad_to(feat_values.astype(jnp.float32), PFEAT_)

    adj_dst_h = adj_dst.reshape(TOTCH_ADJ_ // NB_ADJ_, NB_ADJ_, E_)
    adj_src_h = adj_src
    adj_val_h = adj_val
    feat_dst_h = f_dst.reshape(TOTCH_FEAT_ // NB_FEAT_, NB_FEAT_, E_)
    feat_src_h = f_src
    feat_val_h = f_val

    part0 = _scatter_feat(feat_dst_h, feat_src_h, feat_val_h, weight)
    base1 = _combine(part0, bias)
    part1 = _scatter_adj(adj_dst_h, adj_src_h, adj_val_h, base1)
    base2 = _combine(part1)
    part2 = _scatter_adj(adj_dst_h, adj_src_h, adj_val_h, base2)
    out = _combine(part2)
    return out[:N_NODES_]


# speedup vs baseline: 2.4545x; 2.4545x over previous
import jax
import jax.numpy as jnp
import numpy as np
from jax import lax
from jax.experimental import pallas as pl
from jax.experimental.pallas import tpu as pltpu
from jax.experimental.pallas import tpu_sc as plsc

N_NODES_ = 10000
CH_ = 128
R_ = 10240
ROWS_PER_TILE_ = R_ // 16
E_ = 128
ADJ0_, ADJ1_, NB_ADJ_ = 80, 80, 8
FEAT0_, FEAT1_, NB_FEAT_ = 25, 25, 5
TOTCH_ADJ_ = 16 * (ADJ0_ + ADJ1_)
TOTCH_FEAT_ = 16 * (FEAT0_ + FEAT1_)
PADJ_ = TOTCH_ADJ_ * E_
PFEAT_ = TOTCH_FEAT_ * E_

_GDN = lax.GatherDimensionNumbers(
    offset_dims=(), collapsed_slice_dims=(0,), start_index_map=(0,))


def _lane_bcast(vv, zidx, i):
    return lax.gather(vv, zidx + i, _GDN, slice_sizes=(1,),
                      mode=lax.GatherScatterMode.PROMISE_IN_BOUNDS)


def _make_scatter_body(nch0, nch1, nb):
    def body(dst_h, src_h, val_h, table_h, part_h,
             acc, dstb, srcb, valb, rows0, rows1, sem0, sem1):
        rows_bufs = (rows0, rows1)
        sems = (sem0, sem1)
        c = lax.axis_index("c")
        s = lax.axis_index("s")
        cR = c * R_
        slab0 = s * ROWS_PER_TILE_
        zidx = jnp.reshape(lax.iota(jnp.int32, 16) * 0, (16, 1))
        zv = jnp.zeros((16,), jnp.float32)
        nbat = jnp.where(c == 0, nch0 // nb, nch1 // nb)
        batch0 = jnp.where(c == 0, 0, 16 * (nch0 // nb)) + s * nbat

        def _zb(i, carry):
            for j in range(8):
                rows0[i, pl.ds(j * 16, 16)] = zv
            return carry

        with jax.named_scope("zero_acc"):
            lax.fori_loop(0, E_, _zb, 0)
            for b in range(ROWS_PER_TILE_ // E_):
                pltpu.sync_copy(rows0, acc.at[pl.ds(slab0 + b * E_, E_)])
            plsc.subcore_barrier()

        def _batch(bi, carry):
            bidx = batch0 + bi
            e0 = bidx * (nb * E_)
            pltpu.sync_copy(dst_h.at[bidx], dstb)
            pltpu.sync_copy(src_h.at[pl.ds(e0, nb * E_)], srcb)
            pltpu.sync_copy(val_h.at[pl.ds(e0, nb * E_)], valb)
            h = pltpu.async_copy(table_h.at[srcb.at[pl.ds(0, E_)]],
                                 rows_bufs[0], sems[0])
            for kk in range(nb):
                rows = rows_bufs[kk % 2]
                if kk + 1 < nb:
                    idx_view = srcb.at[pl.ds((kk + 1) * E_, E_)]
                    h_next = pltpu.async_copy(table_h.at[idx_view],
                                              rows_bufs[(kk + 1) % 2],
                                              sems[(kk + 1) % 2])
                h.wait()
                if kk + 1 < nb:
                    h = h_next

                def _scale(g, c2, _kk=kk, _rows=rows):
                    vv = valb[pl.ds(_kk * E_ + g * 16, 16)]
                    for i in range(16):
                        bv = _lane_bcast(vv, zidx, i)
                        e = g * 16 + i
                        for j in range(8):
                            sl = pl.ds(j * 16, 16)
                            _rows[e, sl] = _rows[e, sl] * bv
                    return c2

                lax.fori_loop(0, E_ // 16, _scale, 0)
                pltpu.sync_copy(rows, acc.at[dstb.at[kk]], add=True)
            return carry

        with jax.named_scope("edge_stream"):
            lax.fori_loop(0, nbat, _batch, 0)
        with jax.named_scope("post_barrier"):
            plsc.subcore_barrier()

        with jax.named_scope("dump"):
            pltpu.sync_copy(acc.at[pl.ds(slab0, ROWS_PER_TILE_)],
                            part_h.at[pl.ds(cR + slab0, ROWS_PER_TILE_)])

    return body


def _make_scatter_call(nch0, nch1, nb):
    return pl.kernel(
        _make_scatter_body(nch0, nch1, nb),
        mesh=plsc.VectorSubcoreMesh(core_axis_name="c", subcore_axis_name="s"),
        out_type=jax.ShapeDtypeStruct((2 * R_, CH_), jnp.float32),
        scratch_types=[
            pltpu.VMEM_SHARED((R_, CH_), jnp.float32),
            pltpu.VMEM((nb, E_), jnp.int32),
            pltpu.VMEM((nb * E_,), jnp.int32),
            pltpu.VMEM((nb * E_,), jnp.float32),
            pltpu.VMEM((E_, CH_), jnp.float32),
            pltpu.VMEM((E_, CH_), jnp.float32),
            pltpu.SemaphoreType.DMA,
            pltpu.SemaphoreType.DMA,
        ],
    )


_scatter_feat = _make_scatter_call(FEAT0_, FEAT1_, NB_FEAT_)
_scatter_adj = _make_scatter_call(ADJ0_, ADJ1_, NB_ADJ_)

_BLK_ = 1280


def _combine_relu_body(p0, p1, b, o):
    o[...] = jnp.maximum(p0[...] + p1[...] + b[...], 0.0)


def _combine_body(p0, p1, o):
    o[...] = p0[...] + p1[...]


def _combine(part, bias=None):
    grid = (R_ // _BLK_,)
    spec0 = pl.BlockSpec((_BLK_, CH_), lambda i: (i, 0))
    spec1 = pl.BlockSpec((_BLK_, CH_), lambda i: (R_ // _BLK_ + i, 0))
    if bias is not None:
        return pl.pallas_call(
            _combine_relu_body,
            grid=grid,
            in_specs=[spec0, spec1, pl.BlockSpec((1, CH_), lambda i: (0, 0))],
            out_specs=pl.BlockSpec((_BLK_, CH_), lambda i: (i, 0)),
            out_shape=jax.ShapeDtypeStruct((R_, CH_), jnp.float32),
        )(part, part, bias)
    return pl.pallas_call(
        _combine_body,
        grid=grid,
        in_specs=[spec0, spec1],
        out_specs=pl.BlockSpec((_BLK_, CH_), lambda i: (i, 0)),
        out_shape=jax.ShapeDtypeStruct((R_, CH_), jnp.float32),
    )(part, part)


def _pad_to(x, n, fill=0):
    return jnp.concatenate([x, jnp.full((n - x.shape[0],), fill, x.dtype)])


def _pad_spread(x, n, lo, hi):
    m = n - x.shape[0]
    fill = lo + (np.arange(m, dtype=np.int32) % (hi - lo))
    return jnp.concatenate([x, jnp.asarray(fill)])


@jax.jit
def kernel(adj_indices, adj_values, feat_row, feat_col, feat_values,
           weight, bias):
    adj_dst = _pad_spread(adj_indices[0].astype(jnp.int32), PADJ_, N_NODES_, R_)
    adj_src = _pad_spread(adj_indices[1].astype(jnp.int32), PADJ_, 0, N_NODES_)
    adj_val = _pad_to(adj_values.astype(jnp.float32), PADJ_)
    f_dst = _pad_spread(feat_row.astype(jnp.int32), PFEAT_, N_NODES_, R_)
    f_src = _pad_spread(feat_col.astype(jnp.int32), PFEAT_, 0, CH_)
    f_val = _pad_to(feat_values.astype(jnp.float32), PFEAT_)

    adj_dst_h = adj_dst.reshape(TOTCH_ADJ_ // NB_ADJ_, NB_ADJ_, E_)
    adj_src_h = adj_src
    adj_val_h = adj_val
    feat_dst_h = f_dst.reshape(TOTCH_FEAT_ // NB_FEAT_, NB_FEAT_, E_)
    feat_src_h = f_src
    feat_val_h = f_val

    part0 = _scatter_feat(feat_dst_h, feat_src_h, feat_val_h, weight)
    base1 = _combine(part0, bias)
    part1 = _scatter_adj(adj_dst_h, adj_src_h, adj_val_h, base1)
    base2 = _combine(part1)
    part2 = _scatter_adj(adj_dst_h, adj_src_h, adj_val_h, base2)
    out = _combine(part2)
    return out[:N_NODES_]


# speedup vs baseline: 2.6160x; 1.0658x over previous
import jax
import jax.numpy as jnp
import numpy as np
from jax import lax
from jax.experimental import pallas as pl
from jax.experimental.pallas import tpu as pltpu
from jax.experimental.pallas import tpu_sc as plsc

N_NODES_ = 10000
CH_ = 128
R_ = 10240
ROWS_PER_TILE_ = R_ // 16
E_ = 128
ADJ0_, ADJ1_, NB_ADJ_ = 80, 80, 16
FEAT0_, FEAT1_, NB_FEAT_ = 25, 25, 5
TOTCH_ADJ_ = 16 * (ADJ0_ + ADJ1_)
TOTCH_FEAT_ = 16 * (FEAT0_ + FEAT1_)
PADJ_ = TOTCH_ADJ_ * E_
PFEAT_ = TOTCH_FEAT_ * E_

_GDN = lax.GatherDimensionNumbers(
    offset_dims=(), collapsed_slice_dims=(0,), start_index_map=(0,))


def _lane_bcast(vv, zidx, i):
    return lax.gather(vv, zidx + i, _GDN, slice_sizes=(1,),
                      mode=lax.GatherScatterMode.PROMISE_IN_BOUNDS)


def _make_scatter_body(nch0, nch1, nb):
    def body(dst_h, src_h, val_h, table_h, part_h,
             acc, dstb, srcb, valb, rows0, rows1, sem0, sem1, ssem0, ssem1):
        rows_bufs = (rows0, rows1)
        sems = (sem0, sem1)
        ssems = (ssem0, ssem1)
        c = lax.axis_index("c")
        s = lax.axis_index("s")
        cR = c * R_
        slab0 = s * ROWS_PER_TILE_
        zidx = jnp.reshape(lax.iota(jnp.int32, 16) * 0, (16, 1))
        zv = jnp.zeros((16,), jnp.float32)
        nbat = jnp.where(c == 0, nch0 // nb, nch1 // nb)
        batch0 = jnp.where(c == 0, 0, 16 * (nch0 // nb)) + s * nbat

        def _zb(i, carry):
            for j in range(8):
                rows0[i, pl.ds(j * 16, 16)] = zv
            return carry

        with jax.named_scope("zero_acc"):
            lax.fori_loop(0, E_, _zb, 0)
            for b in range(ROWS_PER_TILE_ // E_):
                pltpu.sync_copy(rows0, acc.at[pl.ds(slab0 + b * E_, E_)])
            plsc.subcore_barrier()

        def _batch(bi, carry):
            bidx = batch0 + bi
            e0 = bidx * (nb * E_)
            pltpu.sync_copy(dst_h.at[bidx], dstb)
            pltpu.sync_copy(src_h.at[pl.ds(e0, nb * E_)], srcb)
            pltpu.sync_copy(val_h.at[pl.ds(e0, nb * E_)], valb)
            h = pltpu.async_copy(table_h.at[srcb.at[pl.ds(0, E_)]],
                                 rows_bufs[0], sems[0])
            sc_h = [None, None]
            for kk in range(nb):
                b = kk % 2
                rows = rows_bufs[b]
                if kk + 1 < nb:
                    nb2 = (kk + 1) % 2
                    if sc_h[nb2] is not None:
                        sc_h[nb2].wait()
                        sc_h[nb2] = None
                    idx_view = srcb.at[pl.ds((kk + 1) * E_, E_)]
                    h_next = pltpu.async_copy(table_h.at[idx_view],
                                              rows_bufs[nb2], sems[nb2])
                h.wait()
                if kk + 1 < nb:
                    h = h_next

                def _scale(g, c2, _kk=kk, _rows=rows):
                    vv = valb[pl.ds(_kk * E_ + g * 16, 16)]
                    for i in range(16):
                        bv = _lane_bcast(vv, zidx, i)
                        e = g * 16 + i
                        for j in range(8):
                            sl = pl.ds(j * 16, 16)
                            _rows[e, sl] = _rows[e, sl] * bv
                    return c2

                lax.fori_loop(0, E_ // 16, _scale, 0)
                sc_h[b] = pltpu.async_copy(rows, acc.at[dstb.at[kk]],
                                           ssems[b], add=True)
            for b in range(2):
                if sc_h[b] is not None:
                    sc_h[b].wait()
            return carry

        with jax.named_scope("edge_stream"):
            lax.fori_loop(0, nbat, _batch, 0)
        with jax.named_scope("post_barrier"):
            plsc.subcore_barrier()

        with jax.named_scope("dump"):
            pltpu.sync_copy(acc.at[pl.ds(slab0, ROWS_PER_TILE_)],
                            part_h.at[pl.ds(cR + slab0, ROWS_PER_TILE_)])

    return body


def _make_scatter_call(nch0, nch1, nb):
    return pl.kernel(
        _make_scatter_body(nch0, nch1, nb),
        mesh=plsc.VectorSubcoreMesh(core_axis_name="c", subcore_axis_name="s"),
        out_type=jax.ShapeDtypeStruct((2 * R_, CH_), jnp.float32),
        scratch_types=[
            pltpu.VMEM_SHARED((R_, CH_), jnp.float32),
            pltpu.VMEM((nb, E_), jnp.int32),
            pltpu.VMEM((nb * E_,), jnp.int32),
            pltpu.VMEM((nb * E_,), jnp.float32),
            pltpu.VMEM((E_, CH_), jnp.float32),
            pltpu.VMEM((E_, CH_), jnp.float32),
            pltpu.SemaphoreType.DMA,
            pltpu.SemaphoreType.DMA,
            pltpu.SemaphoreType.DMA,
            pltpu.SemaphoreType.DMA,
        ],
    )


_scatter_feat = _make_scatter_call(FEAT0_, FEAT1_, NB_FEAT_)
_scatter_adj = _make_scatter_call(ADJ0_, ADJ1_, NB_ADJ_)

_BLK_ = 1280


def _combine_relu_body(p0, p1, b, o):
    o[...] = jnp.maximum(p0[...] + p1[...] + b[...], 0.0)


def _combine_body(p0, p1, o):
    o[...] = p0[...] + p1[...]


def _combine(part, bias=None):
    grid = (R_ // _BLK_,)
    spec0 = pl.BlockSpec((_BLK_, CH_), lambda i: (i, 0))
    spec1 = pl.BlockSpec((_BLK_, CH_), lambda i: (R_ // _BLK_ + i, 0))
    if bias is not None:
        return pl.pallas_call(
            _combine_relu_body,
            grid=grid,
            in_specs=[spec0, spec1, pl.BlockSpec((1, CH_), lambda i: (0, 0))],
            out_specs=pl.BlockSpec((_BLK_, CH_), lambda i: (i, 0)),
            out_shape=jax.ShapeDtypeStruct((R_, CH_), jnp.float32),
        )(part, part, bias)
    return pl.pallas_call(
        _combine_body,
        grid=grid,
        in_specs=[spec0, spec1],
        out_specs=pl.BlockSpec((_BLK_, CH_), lambda i: (i, 0)),
        out_shape=jax.ShapeDtypeStruct((R_, CH_), jnp.float32),
    )(part, part)


def _pad_to(x, n, fill=0):
    return jnp.concatenate([x, jnp.full((n - x.shape[0],), fill, x.dtype)])


def _pad_spread(x, n, lo, hi):
    m = n - x.shape[0]
    fill = lo + (np.arange(m, dtype=np.int32) % (hi - lo))
    return jnp.concatenate([x, jnp.asarray(fill)])


@jax.jit
def kernel(adj_indices, adj_values, feat_row, feat_col, feat_values,
           weight, bias):
    adj_dst = _pad_spread(adj_indices[0].astype(jnp.int32), PADJ_, N_NODES_, R_)
    adj_src = _pad_spread(adj_indices[1].astype(jnp.int32), PADJ_, 0, N_NODES_)
    adj_val = _pad_to(adj_values.astype(jnp.float32), PADJ_)
    f_dst = _pad_spread(feat_row.astype(jnp.int32), PFEAT_, N_NODES_, R_)
    f_src = _pad_spread(feat_col.astype(jnp.int32), PFEAT_, 0, CH_)
    f_val = _pad_to(feat_values.astype(jnp.float32), PFEAT_)

    adj_dst_h = adj_dst.reshape(TOTCH_ADJ_ // NB_ADJ_, NB_ADJ_, E_)
    adj_src_h = adj_src
    adj_val_h = adj_val
    feat_dst_h = f_dst.reshape(TOTCH_FEAT_ // NB_FEAT_, NB_FEAT_, E_)
    feat_src_h = f_src
    feat_val_h = f_val

    part0 = _scatter_feat(feat_dst_h, feat_src_h, feat_val_h, weight)
    base1 = _combine(part0, bias)
    part1 = _scatter_adj(adj_dst_h, adj_src_h, adj_val_h, base1)
    base2 = _combine(part1)
    part2 = _scatter_adj(adj_dst_h, adj_src_h, adj_val_h, base2)
    out = _combine(part2)
    return out[:N_NODES_]


# speedup vs baseline: 2.6217x; 1.0022x over previous
import jax
import jax.numpy as jnp
import numpy as np
from jax import lax
from jax.experimental import pallas as pl
from jax.experimental.pallas import tpu as pltpu
from jax.experimental.pallas import tpu_sc as plsc

N_NODES_ = 10000
CH_ = 128
R_ = 10240
ROWS_PER_TILE_ = R_ // 16
E_ = 128
ADJ0_, ADJ1_, NB_ADJ_ = 80, 80, 16
FEAT0_, FEAT1_, NB_FEAT_ = 25, 25, 25
TOTCH_ADJ_ = 16 * (ADJ0_ + ADJ1_)
TOTCH_FEAT_ = 16 * (FEAT0_ + FEAT1_)
PADJ_ = TOTCH_ADJ_ * E_
PFEAT_ = TOTCH_FEAT_ * E_

_GDN = lax.GatherDimensionNumbers(
    offset_dims=(), collapsed_slice_dims=(0,), start_index_map=(0,))


def _lane_bcast(vv, zidx, i):
    return lax.gather(vv, zidx + i, _GDN, slice_sizes=(1,),
                      mode=lax.GatherScatterMode.PROMISE_IN_BOUNDS)


def _make_scatter_body(nch0, nch1, nb):
    def body(dst_h, src_h, val_h, table_h, part_h,
             acc, dstb, srcb, valb, rows0, rows1, sem0, sem1, ssem0, ssem1):
        rows_bufs = (rows0, rows1)
        sems = (sem0, sem1)
        ssems = (ssem0, ssem1)
        c = lax.axis_index("c")
        s = lax.axis_index("s")
        cR = c * R_
        slab0 = s * ROWS_PER_TILE_
        zidx = jnp.reshape(lax.iota(jnp.int32, 16) * 0, (16, 1))
        zv = jnp.zeros((16,), jnp.float32)
        nbat = jnp.where(c == 0, nch0 // nb, nch1 // nb)
        batch0 = jnp.where(c == 0, 0, 16 * (nch0 // nb)) + s * nbat

        def _zb(i, carry):
            for j in range(8):
                rows0[i, pl.ds(j * 16, 16)] = zv
            return carry

        with jax.named_scope("zero_acc"):
            lax.fori_loop(0, E_, _zb, 0)
            for b in range(ROWS_PER_TILE_ // E_):
                pltpu.sync_copy(rows0, acc.at[pl.ds(slab0 + b * E_, E_)])
            plsc.subcore_barrier()

        def _batch(bi, carry):
            bidx = batch0 + bi
            e0 = bidx * (nb * E_)
            pltpu.sync_copy(dst_h.at[bidx], dstb)
            pltpu.sync_copy(src_h.at[pl.ds(e0, nb * E_)], srcb)
            pltpu.sync_copy(val_h.at[pl.ds(e0, nb * E_)], valb)
            h = pltpu.async_copy(table_h.at[srcb.at[pl.ds(0, E_)]],
                                 rows_bufs[0], sems[0])
            sc_h = [None, None]
            for kk in range(nb):
                b = kk % 2
                rows = rows_bufs[b]
                if kk + 1 < nb:
                    nb2 = (kk + 1) % 2
                    if sc_h[nb2] is not None:
                        sc_h[nb2].wait()
                        sc_h[nb2] = None
                    idx_view = srcb.at[pl.ds((kk + 1) * E_, E_)]
                    h_next = pltpu.async_copy(table_h.at[idx_view],
                                              rows_bufs[nb2], sems[nb2])
                h.wait()
                if kk + 1 < nb:
                    h = h_next

                def _scale(g, c2, _kk=kk, _rows=rows):
                    vv = valb[pl.ds(_kk * E_ + g * 16, 16)]
                    for i in range(16):
                        bv = _lane_bcast(vv, zidx, i)
                        e = g * 16 + i
                        for j in range(8):
                            sl = pl.ds(j * 16, 16)
                            _rows[e, sl] = _rows[e, sl] * bv
                    return c2

                lax.fori_loop(0, E_ // 16, _scale, 0)
                sc_h[b] = pltpu.async_copy(rows, acc.at[dstb.at[kk]],
                                           ssems[b], add=True)
            for b in range(2):
                if sc_h[b] is not None:
                    sc_h[b].wait()
            return carry

        with jax.named_scope("edge_stream"):
            lax.fori_loop(0, nbat, _batch, 0)
        with jax.named_scope("post_barrier"):
            plsc.subcore_barrier()

        with jax.named_scope("dump"):
            pltpu.sync_copy(acc.at[pl.ds(slab0, ROWS_PER_TILE_)],
                            part_h.at[pl.ds(cR + slab0, ROWS_PER_TILE_)])

    return body


def _make_scatter_call(nch0, nch1, nb):
    return pl.kernel(
        _make_scatter_body(nch0, nch1, nb),
        mesh=plsc.VectorSubcoreMesh(core_axis_name="c", subcore_axis_name="s"),
        out_type=jax.ShapeDtypeStruct((2 * R_, CH_), jnp.float32),
        scratch_types=[
            pltpu.VMEM_SHARED((R_, CH_), jnp.float32),
            pltpu.VMEM((nb, E_), jnp.int32),
            pltpu.VMEM((nb * E_,), jnp.int32),
            pltpu.VMEM((nb * E_,), jnp.float32),
            pltpu.VMEM((E_, CH_), jnp.float32),
            pltpu.VMEM((E_, CH_), jnp.float32),
            pltpu.SemaphoreType.DMA,
            pltpu.SemaphoreType.DMA,
            pltpu.SemaphoreType.DMA,
            pltpu.SemaphoreType.DMA,
        ],
    )


_scatter_feat = _make_scatter_call(FEAT0_, FEAT1_, NB_FEAT_)
_scatter_adj = _make_scatter_call(ADJ0_, ADJ1_, NB_ADJ_)

_BLK_ = 1280


def _combine_relu_body(p0, p1, b, o):
    o[...] = jnp.maximum(p0[...] + p1[...] + b[...], 0.0)


def _combine_body(p0, p1, o):
    o[...] = p0[...] + p1[...]


def _combine(part, bias=None):
    grid = (R_ // _BLK_,)
    spec0 = pl.BlockSpec((_BLK_, CH_), lambda i: (i, 0))
    spec1 = pl.BlockSpec((_BLK_, CH_), lambda i: (R_ // _BLK_ + i, 0))
    if bias is not None:
        return pl.pallas_call(
            _combine_relu_body,
            grid=grid,
            in_specs=[spec0, spec1, pl.BlockSpec((1, CH_), lambda i: (0, 0))],
            out_specs=pl.BlockSpec((_BLK_, CH_), lambda i: (i, 0)),
            out_shape=jax.ShapeDtypeStruct((R_, CH_), jnp.float32),
        )(part, part, bias)
    return pl.pallas_call(
        _combine_body,
        grid=grid,
        in_specs=[spec0, spec1],
        out_specs=pl.BlockSpec((_BLK_, CH_), lambda i: (i, 0)),
        out_shape=jax.ShapeDtypeStruct((R_, CH_), jnp.float32),
    )(part, part)


def _pad_to(x, n, fill=0):
    return jnp.concatenate([x, jnp.full((n - x.shape[0],), fill, x.dtype)])


def _pad_spread(x, n, lo, hi):
    m = n - x.shape[0]
    fill = lo + (np.arange(m, dtype=np.int32) % (hi - lo))
    return jnp.concatenate([x, jnp.asarray(fill)])


@jax.jit
def kernel(adj_indices, adj_values, feat_row, feat_col, feat_values,
           weight, bias):
    adj_dst = _pad_spread(adj_indices[0].astype(jnp.int32), PADJ_, N_NODES_, R_)
    adj_src = _pad_spread(adj_indices[1].astype(jnp.int32), PADJ_, 0, N_NODES_)
    adj_val = _pad_to(adj_values.astype(jnp.float32), PADJ_)
    f_dst = _pad_spread(feat_row.astype(jnp.int32), PFEAT_, N_NODES_, R_)
    f_src = _pad_spread(feat_col.astype(jnp.int32), PFEAT_, 0, CH_)
    f_val = _pad_to(feat_values.astype(jnp.float32), PFEAT_)

    adj_dst_h = adj_dst.reshape(TOTCH_ADJ_ // NB_ADJ_, NB_ADJ_, E_)
    adj_src_h = adj_src
    adj_val_h = adj_val
    feat_dst_h = f_dst.reshape(TOTCH_FEAT_ // NB_FEAT_, NB_FEAT_, E_)
    feat_src_h = f_src
    feat_val_h = f_val

    part0 = _scatter_feat(feat_dst_h, feat_src_h, feat_val_h, weight)
    base1 = _combine(part0, bias)
    part1 = _scatter_adj(adj_dst_h, adj_src_h, adj_val_h, base1)
    base2 = _combine(part1)
    part2 = _scatter_adj(adj_dst_h, adj_src_h, adj_val_h, base2)
    out = _combine(part2)
    return out[:N_NODES_]


# speedup vs baseline: 2.7617x; 1.0534x over previous
import jax
import jax.numpy as jnp
import numpy as np
from jax import lax
from jax.experimental import pallas as pl
from jax.experimental.pallas import tpu as pltpu
from jax.experimental.pallas import tpu_sc as plsc

N_NODES_ = 10000
CH_ = 128
R_ = 10240
ROWS_PER_TILE_ = R_ // 16
E_ = 128
ADJ0_, ADJ1_, NB_ADJ_ = 80, 80, 16
FEAT0_, FEAT1_, NB_FEAT_ = 25, 25, 25
WREP_ = 8
TOTCH_ADJ_ = 16 * (ADJ0_ + ADJ1_)
TOTCH_FEAT_ = 16 * (FEAT0_ + FEAT1_)
PADJ_ = TOTCH_ADJ_ * E_
PFEAT_ = TOTCH_FEAT_ * E_

_GDN = lax.GatherDimensionNumbers(
    offset_dims=(), collapsed_slice_dims=(0,), start_index_map=(0,))


def _lane_bcast(vv, zidx, i):
    return lax.gather(vv, zidx + i, _GDN, slice_sizes=(1,),
                      mode=lax.GatherScatterMode.PROMISE_IN_BOUNDS)


def _make_scatter_body(nch0, nch1, nb):
    def body(dst_h, src_h, val_h, table_h, part_h,
             acc, dstb, srcb, valb, rows0, rows1, sem0, sem1, ssem0, ssem1):
        rows_bufs = (rows0, rows1)
        sems = (sem0, sem1)
        ssems = (ssem0, ssem1)
        c = lax.axis_index("c")
        s = lax.axis_index("s")
        cR = c * R_
        slab0 = s * ROWS_PER_TILE_
        zidx = jnp.reshape(lax.iota(jnp.int32, 16) * 0, (16, 1))
        zv = jnp.zeros((16,), jnp.float32)
        nbat = jnp.where(c == 0, nch0 // nb, nch1 // nb)
        batch0 = jnp.where(c == 0, 0, 16 * (nch0 // nb)) + s * nbat

        def _zb(i, carry):
            for j in range(8):
                rows0[i, pl.ds(j * 16, 16)] = zv
            return carry

        with jax.named_scope("zero_acc"):
            lax.fori_loop(0, E_, _zb, 0)
            for b in range(ROWS_PER_TILE_ // E_):
                pltpu.sync_copy(rows0, acc.at[pl.ds(slab0 + b * E_, E_)])
            plsc.subcore_barrier()

        def _batch(bi, carry):
            bidx = batch0 + bi
            e0 = bidx * (nb * E_)
            pltpu.sync_copy(dst_h.at[bidx], dstb)
            pltpu.sync_copy(src_h.at[pl.ds(e0, nb * E_)], srcb)
            pltpu.sync_copy(val_h.at[pl.ds(e0, nb * E_)], valb)
            h = pltpu.async_copy(table_h.at[srcb.at[pl.ds(0, E_)]],
                                 rows_bufs[0], sems[0])
            sc_h = [None, None]
            for kk in range(nb):
                b = kk % 2
                rows = rows_bufs[b]
                if kk + 1 < nb:
                    nb2 = (kk + 1) % 2
                    if sc_h[nb2] is not None:
                        sc_h[nb2].wait()
                        sc_h[nb2] = None
                    idx_view = srcb.at[pl.ds((kk + 1) * E_, E_)]
                    h_next = pltpu.async_copy(table_h.at[idx_view],
                                              rows_bufs[nb2], sems[nb2])
                h.wait()
                if kk + 1 < nb:
                    h = h_next

                def _scale(g, c2, _kk=kk, _rows=rows):
                    vv = valb[pl.ds(_kk * E_ + g * 16, 16)]
                    for i in range(16):
                        bv = _lane_bcast(vv, zidx, i)
                        e = g * 16 + i
                        for j in range(8):
                            sl = pl.ds(j * 16, 16)
                            _rows[e, sl] = _rows[e, sl] * bv
                    return c2

                lax.fori_loop(0, E_ // 16, _scale, 0)
                sc_h[b] = pltpu.async_copy(rows, acc.at[dstb.at[kk]],
                                           ssems[b], add=True)
            for b in range(2):
                if sc_h[b] is not None:
                    sc_h[b].wait()
            return carry

        with jax.named_scope("edge_stream"):
            lax.fori_loop(0, nbat, _batch, 0)
        with jax.named_scope("post_barrier"):
            plsc.subcore_barrier()

        with jax.named_scope("dump"):
            pltpu.sync_copy(acc.at[pl.ds(slab0, ROWS_PER_TILE_)],
                            part_h.at[pl.ds(cR + slab0, ROWS_PER_TILE_)])

    return body


def _make_scatter_call(nch0, nch1, nb):
    return pl.kernel(
        _make_scatter_body(nch0, nch1, nb),
        mesh=plsc.VectorSubcoreMesh(core_axis_name="c", subcore_axis_name="s"),
        out_type=jax.ShapeDtypeStruct((2 * R_, CH_), jnp.float32),
        scratch_types=[
            pltpu.VMEM_SHARED((R_, CH_), jnp.float32),
            pltpu.VMEM((nb, E_), jnp.int32),
            pltpu.VMEM((nb * E_,), jnp.int32),
            pltpu.VMEM((nb * E_,), jnp.float32),
            pltpu.VMEM((E_, CH_), jnp.float32),
            pltpu.VMEM((E_, CH_), jnp.float32),
            pltpu.SemaphoreType.DMA,
            pltpu.SemaphoreType.DMA,
            pltpu.SemaphoreType.DMA,
            pltpu.SemaphoreType.DMA,
        ],
    )


_scatter_feat = _make_scatter_call(FEAT0_, FEAT1_, NB_FEAT_)
_scatter_adj = _make_scatter_call(ADJ0_, ADJ1_, NB_ADJ_)

_BLK_ = 1280


def _combine_relu_body(p0, p1, b, o):
    o[...] = jnp.maximum(p0[...] + p1[...] + b[...], 0.0)


def _combine_body(p0, p1, o):
    o[...] = p0[...] + p1[...]


def _combine(part, bias=None):
    grid = (R_ // _BLK_,)
    spec0 = pl.BlockSpec((_BLK_, CH_), lambda i: (i, 0))
    spec1 = pl.BlockSpec((_BLK_, CH_), lambda i: (R_ // _BLK_ + i, 0))
    if bias is not None:
        return pl.pallas_call(
            _combine_relu_body,
            grid=grid,
            in_specs=[spec0, spec1, pl.BlockSpec((1, CH_), lambda i: (0, 0))],
            out_specs=pl.BlockSpec((_BLK_, CH_), lambda i: (i, 0)),
            out_shape=jax.ShapeDtypeStruct((R_, CH_), jnp.float32),
        )(part, part, bias)
    return pl.pallas_call(
        _combine_body,
        grid=grid,
        in_specs=[spec0, spec1],
        out_specs=pl.BlockSpec((_BLK_, CH_), lambda i: (i, 0)),
        out_shape=jax.ShapeDtypeStruct((R_, CH_), jnp.float32),
    )(part, part)


def _pad_to(x, n, fill=0):
    return jnp.concatenate([x, jnp.full((n - x.shape[0],), fill, x.dtype)])


def _pad_spread(x, n, lo, hi):
    m = n - x.shape[0]
    fill = lo + (np.arange(m, dtype=np.int32) % (hi - lo))
    return jnp.concatenate([x, jnp.asarray(fill)])


@jax.jit
def kernel(adj_indices, adj_values, feat_row, feat_col, feat_values,
           weight, bias):
    adj_dst = _pad_spread(adj_indices[0].astype(jnp.int32), PADJ_, N_NODES_, R_)
    adj_src = _pad_spread(adj_indices[1].astype(jnp.int32), PADJ_, 0, N_NODES_)
    adj_val = _pad_to(adj_values.astype(jnp.float32), PADJ_)
    f_dst = _pad_spread(feat_row.astype(jnp.int32), PFEAT_, N_NODES_, R_)
    nfc = feat_col.shape[0]
    f_src = feat_col.astype(jnp.int32) + CH_ * jnp.asarray(
        np.arange(nfc, dtype=np.int32) % WREP_)
    f_src = _pad_spread(f_src, PFEAT_, 0, WREP_ * CH_)
    f_val = _pad_to(feat_values.astype(jnp.float32), PFEAT_)

    adj_dst_h = adj_dst.reshape(TOTCH_ADJ_ // NB_ADJ_, NB_ADJ_, E_)
    adj_src_h = adj_src
    adj_val_h = adj_val
    feat_dst_h = f_dst.reshape(TOTCH_FEAT_ // NB_FEAT_, NB_FEAT_, E_)
    feat_src_h = f_src
    feat_val_h = f_val

    weight_rep = jnp.tile(weight, (WREP_, 1))
    part0 = _scatter_feat(feat_dst_h, feat_src_h, feat_val_h, weight_rep)
    base1 = _combine(part0, bias)
    part1 = _scatter_adj(adj_dst_h, adj_src_h, adj_val_h, base1)
    base2 = _combine(part1)
    part2 = _scatter_adj(adj_dst_h, adj_src_h, adj_val_h, base2)
    out = _combine(part2)
    return out[:N_NODES_]
